# baseline (speedup 1.0000x reference)
import functools

import jax
import jax.numpy as jnp
from jax import lax
from jax.experimental import pallas as pl
from jax.experimental.pallas import tpu as pltpu
from jax.experimental.pallas import tpu_sc as plsc

_L = 2048
_D = 64
_NW = 32
_RPW = _L // _NW


def _sc_body(table_hbm, out_hbm, tvx, sem):
    nc = 2
    wid = lax.axis_index("s") * nc + lax.axis_index("c")
    base = wid * _RPW

    pltpu.sync_copy(table_hbm, tvx.at[pl.ds(384, 257)])
    t0 = [tvx[384, pl.ds(16 * l, 16)] for l in range(4)]
    t256 = [tvx[640, pl.ds(16 * l, 16)] for l in range(4)]

    def _fill_lo(r, carry):
        for l in range(4):
            tvx[r, pl.ds(16 * l, 16)] = t0[l]
        return carry

    def _fill_hi(r, carry):
        for l in range(4):
            tvx[r, pl.ds(16 * l, 16)] = t256[l]
        return carry

    lax.fori_loop(0, 384, _fill_lo, 0)
    lax.fori_loop(641, 1024, _fill_hi, 0)

    def _row(r, carry):
        i = base + r
        wstart = (jnp.clip(i - 128, 0, 1536) // 256) * 256
        nw = wstart // 256
        copies = [
            pltpu.make_async_copy(
                tvx.at[pl.ds(wstart - i + 512, 512)],
                out_hbm.at[i, pl.ds(wstart, 512)],
                sem,
            )
        ]
        for c in range(6):
            is_pre = c < nw
            dst0 = jnp.where(is_pre, 256 * c, wstart + 512 + 256 * c - 256 * nw)
            src0 = jnp.where(is_pre, 0, 768)
            copies.append(
                pltpu.make_async_copy(
                    tvx.at[pl.ds(src0, 256)],
                    out_hbm.at[i, pl.ds(dst0, 256)],
                    sem,
                )
            )
        for cp in copies:
            cp.start()
        for cp in copies:
            cp.wait()
        return carry

    lax.fori_loop(0, _RPW, _row, 0)


def kernel(embeddings_table, length_q, length_k):
    del length_q, length_k
    run = functools.partial(
        pl.kernel,
        out_type=jax.ShapeDtypeStruct((_L, _L, _D), jnp.float32),
        mesh=plsc.VectorSubcoreMesh(core_axis_name="c", subcore_axis_name="s"),
        scratch_types=[
            pltpu.VMEM((1024, _D), jnp.float32),
            pltpu.SemaphoreType.DMA,
        ],
    )(_sc_body)
    return run(embeddings_table)
